# baseline (speedup 1.0000x reference)
import jax
import jax.numpy as jnp
from jax.experimental import pallas as pl
from jax.experimental.pallas import tpu as pltpu
from jax.experimental.pallas import tpu_sc as plsc

K_TILE = 512
GATHER_W = 128
ROW_SPLIT = 4


def _argmin_tc(x_bf, keys_bf, e_sq, k_sq, b, n, k_total):
    k_tiles = k_total // K_TILE

    def body(x_ref, keys_ref, esq_ref, ksq_ref, out_ref, minval_ref, minidx_ref):
        kt = pl.program_id(0)
        bi = pl.program_id(1)
        xr = x_ref[bi]
        kb = keys_ref[...]
        cross_t = jnp.dot(kb, xr, preferred_element_type=jnp.float32)
        esq = esq_ref[pl.ds(bi, 1), :]
        ksq = ksq_ref[...]
        dist = (esq - 2.0 * cross_t) + ksq
        tmin = jnp.min(dist, axis=0, keepdims=True)
        rows = jax.lax.broadcasted_iota(jnp.int32, dist.shape, 0)
        big = jnp.int32(2**30)
        cand = jnp.min(jnp.where(dist == tmin, rows, big),
                       axis=0, keepdims=True) + kt * K_TILE

        @pl.when(kt == 0)
        def _init():
            minval_ref[pl.ds(bi, 1), :] = tmin
            minidx_ref[pl.ds(bi, 1), :] = cand

        @pl.when(kt > 0)
        def _update():
            old_v = minval_ref[pl.ds(bi, 1), :]
            old_i = minidx_ref[pl.ds(bi, 1), :]
            better = tmin < old_v
            minval_ref[pl.ds(bi, 1), :] = jnp.where(better, tmin, old_v)
            minidx_ref[pl.ds(bi, 1), :] = jnp.where(better, cand, old_i)

        @pl.when(kt == k_tiles - 1)
        def _emit():
            out_ref[pl.ds(bi, 1), :] = minidx_ref[pl.ds(bi, 1), :]

    d = x_bf.shape[1]
    return pl.pallas_call(
        body,
        grid=(k_tiles, b),
        in_specs=[
            pl.BlockSpec((b, d, n), lambda kt, bi: (0, 0, 0)),
            pl.BlockSpec((K_TILE, d), lambda kt, bi: (kt, 0)),
            pl.BlockSpec((b, n), lambda kt, bi: (0, 0)),
            pl.BlockSpec((K_TILE, 1), lambda kt, bi: (kt, 0)),
        ],
        out_specs=pl.BlockSpec((b, n), lambda kt, bi: (0, 0)),
        out_shape=jax.ShapeDtypeStruct((b, n), jnp.int32),
        scratch_shapes=[
            pltpu.VMEM((b, n), jnp.float32),
            pltpu.VMEM((b, n), jnp.int32),
        ],
        compiler_params=pltpu.CompilerParams(
            dimension_semantics=("arbitrary", "arbitrary"),
        ),
    )(x_bf, keys_bf, e_sq, k_sq)


def _gather_sc(values2, idx_flat, d):
    n_tot = idx_flat.shape[0]
    sub_d = d // ROW_SPLIT
    n_sub = n_tot * ROW_SPLIT
    mesh = plsc.VectorSubcoreMesh(core_axis_name="core",
                                  subcore_axis_name="subcore")
    vals_sub = values2.reshape(values2.shape[0] * ROW_SPLIT, sub_d)
    idx_sub = (idx_flat[:, None] * ROW_SPLIT
               + jnp.arange(ROW_SPLIT, dtype=jnp.int32)[None, :])
    idx_sub = idx_sub.reshape(1, n_sub)

    @pl.kernel(out_type=jax.ShapeDtypeStruct((n_sub, sub_d), jnp.float32),
               mesh=mesh)
    def gk(values_hbm, i_hbm, o_hbm):
        def gather_body(i_vmem, o_vmem):
            pltpu.sync_copy(values_hbm.at[i_vmem.at[0]], o_vmem)

        pltpu.emit_pipeline(
            gather_body,
            grid=(n_sub // GATHER_W,),
            in_specs=[pl.BlockSpec((1, GATHER_W), index_map=lambda i: (0, i))],
            out_specs=[pl.BlockSpec((GATHER_W, sub_d), index_map=lambda i: (i, 0))],
            core_axis_name=("core", "subcore"),
            dimension_semantics=(pltpu.PARALLEL,),
        )(i_hbm, o_hbm)

    return gk(vals_sub, idx_sub).reshape(n_tot, d)


def kernel(x, keys, values):
    b, d, s, _ = x.shape
    n = s * s
    k_total = keys.shape[1]

    xr = x.reshape(b, d, n)
    x_bf = xr.astype(jnp.bfloat16)
    keys2 = keys[0]
    keys_bf = keys2.astype(jnp.bfloat16)

    emb = jnp.transpose(xr, (0, 2, 1))
    e_sq = jnp.sum(emb * emb, axis=-1)
    k_sq = jnp.sum(keys2 * keys2, axis=-1)
    k_sq = k_sq.reshape(k_total, 1)

    idx = _argmin_tc(x_bf, keys_bf, e_sq, k_sq, b, n, k_total)

    mem = _gather_sc(values[0], idx.reshape(b * n), d)

    out = jnp.transpose(mem.reshape(b, n, d), (0, 2, 1)).reshape(b, d, s, s)
    return out
